# Initial kernel scaffold; baseline (speedup 1.0000x reference)
import jax
import jax.numpy as jnp
from jax.experimental import pallas as pl


def kernel(adjacency_edge_index, adjacency_values, input_feature, graph_indicator, W1, b1, W2, b2, W3, b3, fc1_W, fc1_b, fc2_W, fc2_b, fc3_W, fc3_b):
    raise NotImplementedError("write your pallas kernel here")


# speedup vs baseline: 1.0448x; 1.0448x over previous
import jax
import jax.numpy as jnp
from jax.experimental import pallas as pl

N = 10000
D = 128
H = 128


def _mm_kernel(x_ref, w_ref, o_ref):
    o_ref[...] = jnp.dot(x_ref[...], w_ref[...],
                         preferred_element_type=jnp.float32)


def _matmul(x, w):
    return pl.pallas_call(
        _mm_kernel,
        out_shape=jax.ShapeDtypeStruct((x.shape[0], w.shape[1]), jnp.float32),
    )(x, w)


def _relu_mm_kernel(p_ref, b_ref, w_ref, g_ref, h_ref):
    g = jnp.maximum(p_ref[...] + b_ref[...], 0.0)
    g_ref[...] = g
    h_ref[...] = jnp.dot(g, w_ref[...], preferred_element_type=jnp.float32)


def _relu_then_matmul(p, b, w):
    return pl.pallas_call(
        _relu_mm_kernel,
        out_shape=(
            jax.ShapeDtypeStruct(p.shape, jnp.float32),
            jax.ShapeDtypeStruct((p.shape[0], w.shape[1]), jnp.float32),
        ),
    )(p, b.reshape(1, -1), w)


def _head_kernel(p_ref, b3_ref, g1_ref, g2_ref, f1w1_ref, f1w2_ref, f1w3_ref,
                 f1b_ref, f2w_ref, f2b_ref, f3w_ref, f3b_ref, o_ref):
    g3 = jnp.maximum(p_ref[...] + b3_ref[...], 0.0)
    f1 = jnp.dot(g1_ref[...], f1w1_ref[...], preferred_element_type=jnp.float32)
    f1 += jnp.dot(g2_ref[...], f1w2_ref[...], preferred_element_type=jnp.float32)
    f1 += jnp.dot(g3, f1w3_ref[...], preferred_element_type=jnp.float32)
    f1 = jnp.maximum(f1 + f1b_ref[...], 0.0)
    f2 = jnp.maximum(jnp.dot(f1, f2w_ref[...], preferred_element_type=jnp.float32)
                     + f2b_ref[...], 0.0)
    o_ref[...] = (jnp.dot(f2, f3w_ref[...], preferred_element_type=jnp.float32)
                  + f3b_ref[...])


def _head(p3, b3, g1, g2, fc1_W, fc1_b, fc2_W, fc2_b, fc3_W, fc3_b):
    return pl.pallas_call(
        _head_kernel,
        out_shape=jax.ShapeDtypeStruct((p3.shape[0], fc3_W.shape[1]),
                                       jnp.float32),
    )(p3, b3.reshape(1, -1), g1, g2,
      fc1_W[:H], fc1_W[H:2 * H], fc1_W[2 * H:],
      fc1_b.reshape(1, -1), fc2_W, fc2_b.reshape(1, -1),
      fc3_W, fc3_b.reshape(1, -1))


def _spmm(row, col, vals, x):
    return jax.ops.segment_sum(vals[:, None] * x[col], row, num_segments=N)


def kernel(adjacency_edge_index, adjacency_values, input_feature,
           graph_indicator, W1, b1, W2, b2, W3, b3,
           fc1_W, fc1_b, fc2_W, fc2_b, fc3_W, fc3_b):
    row = adjacency_edge_index[0]
    col = adjacency_edge_index[1]
    h1 = _matmul(input_feature, W1)
    p1 = _spmm(row, col, adjacency_values, h1)
    g1, h2 = _relu_then_matmul(p1, b1, W2)
    p2 = _spmm(row, col, adjacency_values, h2)
    g2, h3 = _relu_then_matmul(p2, b2, W3)
    p3 = _spmm(row, col, adjacency_values, h3)
    return _head(p3, b3, g1, g2, fc1_W, fc1_b, fc2_W, fc2_b, fc3_W, fc3_b)


# speedup vs baseline: 3.7840x; 3.6216x over previous
import dataclasses
import functools

import jax
import jax.numpy as jnp
from jax import lax
from jax.experimental import pallas as pl
from jax.experimental.pallas import tpu as pltpu
from jax.experimental.pallas import tpu_sc as plsc

N = 10000
E = 320000
D = 128
H = 128

NC = 2
NS = 16
L = 16
CH = 80
EPW = E // (NC * NS)
NPAD = 10240
ROWS_PER_TILE = NPAD // NS
ZR = 128


def _mm_kernel(x_ref, w_ref, o_ref):
    o_ref[...] = jnp.dot(x_ref[...], w_ref[...],
                         preferred_element_type=jnp.float32)


def _matmul(x, w):
    return pl.pallas_call(
        _mm_kernel,
        out_shape=jax.ShapeDtypeStruct((x.shape[0], w.shape[1]), jnp.float32),
    )(x, w)


def _relu_mm_kernel(p_ref, b_ref, w_ref, g_ref, h_ref):
    g = jnp.maximum(p_ref[0, :N] + p_ref[1, :N] + b_ref[...], 0.0)
    g_ref[...] = g
    h_ref[...] = jnp.dot(g, w_ref[...], preferred_element_type=jnp.float32)


def _relu_then_matmul(p, b, w):
    return pl.pallas_call(
        _relu_mm_kernel,
        out_shape=(
            jax.ShapeDtypeStruct((N, p.shape[2]), jnp.float32),
            jax.ShapeDtypeStruct((N, w.shape[1]), jnp.float32),
        ),
    )(p, b.reshape(1, -1), w)


def _head_kernel(p_ref, b3_ref, g1_ref, g2_ref, f1w1_ref, f1w2_ref, f1w3_ref,
                 f1b_ref, f2w_ref, f2b_ref, f3w_ref, f3b_ref, o_ref):
    g3 = jnp.maximum(p_ref[0, :N] + p_ref[1, :N] + b3_ref[...], 0.0)
    f1 = jnp.dot(g1_ref[...], f1w1_ref[...], preferred_element_type=jnp.float32)
    f1 += jnp.dot(g2_ref[...], f1w2_ref[...], preferred_element_type=jnp.float32)
    f1 += jnp.dot(g3, f1w3_ref[...], preferred_element_type=jnp.float32)
    f1 = jnp.maximum(f1 + f1b_ref[...], 0.0)
    f2 = jnp.maximum(jnp.dot(f1, f2w_ref[...], preferred_element_type=jnp.float32)
                     + f2b_ref[...], 0.0)
    o_ref[...] = (jnp.dot(f2, f3w_ref[...], preferred_element_type=jnp.float32)
                  + f3b_ref[...])


def _head(p3, b3, g1, g2, fc1_W, fc1_b, fc2_W, fc2_b, fc3_W, fc3_b):
    return pl.pallas_call(
        _head_kernel,
        out_shape=jax.ShapeDtypeStruct((N, fc3_W.shape[1]),
                                       jnp.float32),
    )(p3, b3.reshape(1, -1), g1, g2,
      fc1_W[:H], fc1_W[H:2 * H], fc1_W[2 * H:],
      fc1_b.reshape(1, -1), fc2_W, fc2_b.reshape(1, -1),
      fc3_W, fc3_b.reshape(1, -1))


def _lane_splat(v, t):
    idx = jnp.full((L, 1), t, jnp.int32)
    dnums = lax.GatherDimensionNumbers(
        offset_dims=(), collapsed_slice_dims=(0,), start_index_map=(0,))
    return lax.gather(v, idx, dnums, (1,),
                      mode=lax.GatherScatterMode.PROMISE_IN_BOUNDS)


def _sc_compiler_params():
    cp = pltpu.CompilerParams()
    if "needs_layout_passes" in pltpu.CompilerParams.__dataclass_fields__:
        cp = dataclasses.replace(cp, needs_layout_passes=False)
    return cp


def _spmm_sc_body(row_hbm, col_hbm, val_hbm, g_hbm, out_hbm,
                  row_v, col_v, val_v, feat_v, zbuf, acc, sem):
    c = lax.axis_index("c")
    s = lax.axis_index("s")
    base = (c * NS + s) * EPW

    @pl.loop(0, ZR)
    def _(i):
        for f in range(D // L):
            zbuf[i, pl.ds(f * L, L)] = jnp.zeros((L,), jnp.float32)

    @pl.loop(0, ROWS_PER_TILE, step=ZR)
    def _(r):
        pltpu.sync_copy(zbuf, acc.at[pl.ds(s * ROWS_PER_TILE + r, ZR)])

    plsc.subcore_barrier()

    @pl.loop(0, EPW, step=CH)
    def _(e0):
        pltpu.sync_copy(row_hbm.at[pl.ds(base + e0, CH)], row_v)
        pltpu.sync_copy(col_hbm.at[pl.ds(base + e0, CH)], col_v)
        pltpu.sync_copy(val_hbm.at[pl.ds(base + e0, CH)], val_v)
        pltpu.async_copy(g_hbm.at[col_v], feat_v, sem).wait()

        @pl.loop(0, CH, step=L)
        def _(j):
            vj = val_v[pl.ds(j, L)]
            for t in range(L):
                vv = _lane_splat(vj, t)
                for f in range(D // L):
                    sl = pl.ds(f * L, L)
                    feat_v[j + t, sl] = feat_v[j + t, sl] * vv

        pltpu.sync_copy(feat_v, acc.at[row_v], add=True)

    plsc.subcore_barrier()

    pltpu.sync_copy(acc.at[pl.ds(s * ROWS_PER_TILE, ROWS_PER_TILE)],
                    out_hbm.at[c].at[pl.ds(s * ROWS_PER_TILE, ROWS_PER_TILE)])


@jax.jit
def _spmm_partials(row, col, vals, g):
    mesh = plsc.VectorSubcoreMesh(core_axis_name="c", subcore_axis_name="s")
    f = pl.kernel(
        _spmm_sc_body,
        out_type=jax.ShapeDtypeStruct((NC, NPAD, D), jnp.float32),
        mesh=mesh,
        scratch_types=[
            pltpu.VMEM((CH,), jnp.int32),
            pltpu.VMEM((CH,), jnp.int32),
            pltpu.VMEM((CH,), jnp.float32),
            pltpu.VMEM((CH, D), jnp.float32),
            pltpu.VMEM((ZR, D), jnp.float32),
            pltpu.VMEM_SHARED((NPAD, D), jnp.float32),
            pltpu.SemaphoreType.DMA,
        ],
        compiler_params=_sc_compiler_params(),
    )
    return f(row, col, vals, g)


def kernel(adjacency_edge_index, adjacency_values, input_feature,
           graph_indicator, W1, b1, W2, b2, W3, b3,
           fc1_W, fc1_b, fc2_W, fc2_b, fc3_W, fc3_b):
    row = adjacency_edge_index[0]
    col = adjacency_edge_index[1]
    h1 = _matmul(input_feature, W1)
    p1 = _spmm_partials(row, col, adjacency_values, h1)
    g1, h2 = _relu_then_matmul(p1, b1, W2)
    p2 = _spmm_partials(row, col, adjacency_values, h2)
    g2, h3 = _relu_then_matmul(p2, b2, W3)
    p3 = _spmm_partials(row, col, adjacency_values, h3)
    return _head(p3, b3, g1, g2, fc1_W, fc1_b, fc2_W, fc2_b, fc3_W, fc3_b)
